# baseline (speedup 1.0000x reference)
import functools

import jax
import jax.numpy as jnp
from jax import lax
from jax.experimental import pallas as pl
from jax.experimental.pallas import tpu as pltpu
from jax.experimental.pallas import tpu_sc as plsc

B, S, D = 4, 8192, 1024
_INFO = plsc.get_sparse_core_info()
NC, NS, L = _INFO.num_cores, _INFO.num_subcores, _INFO.num_lanes
NW = NC * NS
S_PER_W = S // NW
R = 16
VECS_PER_ROW = D // L


def _sc_body(x_hbm, t_hbm, o_hbm, tbuf, xbuf):
    wid = lax.axis_index("s") * NC + lax.axis_index("c")
    s0 = wid * S_PER_W

    def chunk_body(c, _):
        row = s0 + c * R
        pltpu.sync_copy(t_hbm.at[pl.ds(row, R)], tbuf)
        for b in range(B):
            pltpu.sync_copy(x_hbm.at[b, pl.ds(row, R)], xbuf)

            def row_body(r, _):
                def vec_body(jj, _):
                    j = jj * (4 * L)
                    for u in range(4):
                        off = j + u * L
                        xbuf[r, pl.ds(off, L)] = (
                            xbuf[r, pl.ds(off, L)] + tbuf[r, pl.ds(off, L)]
                        )
                    return 0

                return lax.fori_loop(0, VECS_PER_ROW // 4, vec_body, 0)

            lax.fori_loop(0, R, row_body, 0)
            pltpu.sync_copy(xbuf, o_hbm.at[b, pl.ds(row, R)])
        return 0

    lax.fori_loop(0, S_PER_W // R, chunk_body, 0)


def kernel(inputs, emb_table):
    sc_add = functools.partial(
        pl.kernel,
        out_type=jax.ShapeDtypeStruct((B, S, D), jnp.float32),
        mesh=plsc.VectorSubcoreMesh(core_axis_name="c", subcore_axis_name="s"),
        scratch_types=[
            pltpu.VMEM((R, D), jnp.float32),
            pltpu.VMEM((R, D), jnp.float32),
        ],
    )(_sc_body)
    return sc_add(inputs, emb_table)


# speedup vs baseline: 1.0946x; 1.0946x over previous
import functools

import jax
import jax.numpy as jnp
from jax import lax
from jax.experimental import pallas as pl
from jax.experimental.pallas import tpu as pltpu
from jax.experimental.pallas import tpu_sc as plsc

B, S, D = 4, 8192, 1024
_INFO = plsc.get_sparse_core_info()
NC, NS, L = _INFO.num_cores, _INFO.num_subcores, _INFO.num_lanes
NW = NC * NS
S_PER_W = S // NW
R = 16
CHUNK = R * D


def _sc_body(x_hbm, t_hbm, o_hbm, tbuf, xbuf):
    wid = lax.axis_index("s") * NC + lax.axis_index("c")
    base = wid * S_PER_W * D

    def chunk_body(c, _):
        off = base + c * CHUNK
        pltpu.sync_copy(t_hbm.at[pl.ds(off, CHUNK)], tbuf)
        for b in range(B):
            pltpu.sync_copy(x_hbm.at[b, pl.ds(off, CHUNK)], xbuf)

            @plsc.parallel_loop(0, CHUNK, step=L, unroll=8)
            def add_loop(i):
                plsc.addupdate(xbuf.at[pl.ds(i, L)], tbuf[pl.ds(i, L)])

            pltpu.sync_copy(xbuf, o_hbm.at[b, pl.ds(off, CHUNK)])
        return 0

    lax.fori_loop(0, S_PER_W * D // CHUNK, chunk_body, 0)


def kernel(inputs, emb_table):
    sc_add = functools.partial(
        pl.kernel,
        out_type=jax.ShapeDtypeStruct((B, S * D), jnp.float32),
        mesh=plsc.VectorSubcoreMesh(core_axis_name="c", subcore_axis_name="s"),
        scratch_types=[
            pltpu.VMEM((CHUNK,), jnp.float32),
            pltpu.VMEM((CHUNK,), jnp.float32),
        ],
    )(_sc_body)
    out = sc_add(inputs.reshape(B, S * D), emb_table.reshape(S * D))
    return out.reshape(B, S, D)


# speedup vs baseline: 1.4369x; 1.3127x over previous
import functools

import jax
import jax.numpy as jnp
from jax import lax
from jax.experimental import pallas as pl
from jax.experimental.pallas import tpu as pltpu
from jax.experimental.pallas import tpu_sc as plsc

B, S, D = 4, 8192, 1024
_INFO = plsc.get_sparse_core_info()
NC, NS, L = _INFO.num_cores, _INFO.num_subcores, _INFO.num_lanes
NW = NC * NS
S_PER_W = S // NW
R = 8
CHUNK = R * D
W = S_PER_W // R


def _sc_body(x_hbm, t_hbm, o_hbm, bufs, sems):
    xbufs, tbufs = bufs[:8], bufs[8:]
    xsems, tsems = sems[:8], sems[8:]
    wid = lax.axis_index("s") * NC + lax.axis_index("c")
    base = wid * S_PER_W * D

    def start_t(c, p):
        pltpu.async_copy(t_hbm.at[pl.ds(base + c * CHUNK, CHUNK)], tbufs[p], tsems[p])

    def start_in(c, b, i):
        pltpu.async_copy(x_hbm.at[b, pl.ds(base + c * CHUNK, CHUNK)], xbufs[i], xsems[i])

    def start_out(c, b, i):
        pltpu.async_copy(xbufs[i], o_hbm.at[b, pl.ds(base + c * CHUNK, CHUNK)], xsems[i])

    def wait_x(i):
        pltpu.make_async_copy(t_hbm.at[pl.ds(base, CHUNK)], xbufs[i], xsems[i]).wait()

    def wait_t(p):
        pltpu.make_async_copy(t_hbm.at[pl.ds(base, CHUNK)], tbufs[p], tsems[p]).wait()

    def compute(p, i):
        @plsc.parallel_loop(0, CHUNK, step=L, unroll=8)
        def add_loop(v):
            plsc.addupdate(xbufs[i].at[pl.ds(v, L)], tbufs[p][pl.ds(v, L)])

    def process(c, p, refill):
        wait_t(p)
        for b in range(B):
            i = 4 * p + b
            wait_x(i)
            compute(p, i)
            start_out(c, b, i)
        if refill:
            start_t(c + 2, p)

    def refill(c, p):
        for b in range(B):
            i = 4 * p + b
            wait_x(i)
            start_in(c + 2, b, i)

    start_t(0, 0)
    for b in range(B):
        start_in(0, b, b)
    start_t(1, 1)
    for b in range(B):
        start_in(1, b, 4 + b)

    def loop_body(j, _):
        c0 = 2 * j
        process(c0, 0, True)
        process(c0 + 1, 1, True)
        refill(c0, 0)
        refill(c0 + 1, 1)
        return 0

    lax.fori_loop(0, W // 2 - 1, loop_body, 0)

    process(W - 2, 0, False)
    process(W - 1, 1, False)
    for i in range(8):
        wait_x(i)


def kernel(inputs, emb_table):
    sc_add = functools.partial(
        pl.kernel,
        out_type=jax.ShapeDtypeStruct((B, S * D), jnp.float32),
        mesh=plsc.VectorSubcoreMesh(core_axis_name="c", subcore_axis_name="s"),
        scratch_types=[
            [pltpu.VMEM((CHUNK,), jnp.float32) for _ in range(10)],
            [pltpu.SemaphoreType.DMA for _ in range(10)],
        ],
    )(_sc_body)
    out = sc_add(inputs.reshape(B, S * D), emb_table.reshape(S * D))
    return out.reshape(B, S, D)
